# baseline (speedup 1.0000x reference)
import dataclasses
import functools

import jax
import jax.numpy as jnp
from jax import lax
from jax.experimental import pallas as pl
from jax.experimental.pallas import tpu as pltpu
from jax.experimental.pallas import tpu_sc as plsc

NC, NS = 2, 16
NW = NC * NS
B2 = 8192
DIM = 128
CHUNK = 128
PER_W = B2 // NW
NCHUNK = PER_W // CHUNK
NBUF = 4


def _sc_gather_all(tables, idxs, gsel, bounds, perm):
    mesh = plsc.VectorSubcoreMesh(core_axis_name="c", subcore_axis_name="s")
    ng = len(tables)
    out_type = [jax.ShapeDtypeStruct((B2, DIM), jnp.float32)] * ng
    cp = pltpu.CompilerParams()
    if "needs_layout_passes" in pltpu.CompilerParams.__dataclass_fields__:
        cp = dataclasses.replace(cp, needs_layout_passes=False)

    @functools.partial(
        pl.kernel,
        mesh=mesh,
        out_type=out_type,
        compiler_params=cp,
        scratch_types=(
            [pltpu.VMEM((PER_W,), jnp.int32)] * 4
            + [pltpu.VMEM((CHUNK, DIM), jnp.float32)] * NBUF
            + [pltpu.VMEM((16,), jnp.int32)]
            + [pltpu.SemaphoreType.DMA] * (1 + 2 * NBUF)
        ),
    )
    def k(*refs):
        t_refs = refs[:ng]
        i_refs = refs[ng:ng + 3]
        b_ref = refs[ng + 3]
        p_ref = refs[ng + 4]
        o_refs = refs[ng + 5:2 * ng + 5]
        base = 2 * ng + 5
        idx_v = refs[base:base + 3]
        perm_v = refs[base + 3]
        rows_v = refs[base + 4:base + 4 + NBUF]
        bnd_s = refs[base + 4 + NBUF]
        sem_i = refs[base + 5 + NBUF]
        sem_g = refs[base + 6 + NBUF:base + 6 + 2 * NBUF]
        sem_w = refs[base + 6 + 2 * NBUF:base + 6 + 3 * NBUF]
        wid = lax.axis_index("s") * NC + lax.axis_index("c")
        base0 = wid * PER_W

        pltpu.sync_copy(b_ref, bnd_s)
        pltpu.sync_copy(p_ref.at[pl.ds(base0, PER_W)], perm_v)
        pf = [pltpu.make_async_copy(i_refs[a].at[perm_v], idx_v[a], sem_i)
              for a in range(3)]
        for d in pf:
            d.start()
        for d in pf:
            d.wait()

        bvec = bnd_s[...]
        lane = jnp.arange(16, dtype=jnp.int32)
        bs = [jnp.max(jnp.where(lane == j, bvec, 0)) for j in range(4)]

        slots = []
        for g in range(ng):
            a, j = gsel[g]
            for c in range(NCHUNK):
                slots.append((g, a, c * CHUNK, j))

        def cond_of(j, off):
            if j is None:
                return None
            p = base0 + off
            return jnp.logical_and(p < bs[j + 1], p + CHUNK > bs[j])

        def guarded(cond, fn):
            if cond is None:
                fn()
            else:
                pl.when(cond)(fn)

        conds = [cond_of(j, off) for (_, _, off, j) in slots]
        gd = [None] * NBUF
        wd = [None] * NBUF

        def issue_writeback(kk):
            s = kk % NBUF
            g, _, off, _ = slots[kk]
            d, c = gd[s]
            guarded(c, d.wait)
            w = pltpu.make_async_copy(
                rows_v[s], o_refs[g].at[pl.ds(base0 + off, CHUNK)], sem_w[s])
            guarded(c, w.start)
            wd[s] = (w, c)

        for kk in range(len(slots)):
            s = kk % NBUF
            if wd[s] is not None:
                d, c = wd[s]
                guarded(c, d.wait)
                wd[s] = None
            g, a, off, j = slots[kk]
            d = pltpu.make_async_copy(
                t_refs[g].at[idx_v[a].at[pl.ds(off, CHUNK)]],
                rows_v[s], sem_g[s])
            guarded(conds[kk], d.start)
            gd[s] = (d, conds[kk])
            if kk >= NBUF - 1:
                issue_writeback(kk - (NBUF - 1))
        for kk in range(max(0, len(slots) - (NBUF - 1)), len(slots)):
            issue_writeback(kk)
        for s in range(NBUF):
            if wd[s] is not None:
                d, c = wd[s]
                guarded(c, d.wait)

    return k(*tables, *idxs, bounds, perm)


def _tc_score(g, bounds_tc):
    blk = 512

    def body(bnd, h, sh0, sh1, sh2, t, st0, st1, st2, r, sq0, sq1, sq2, o):
        def nrm(x):
            s = jnp.sum(x * x, axis=1, keepdims=True)
            return x * lax.rsqrt(jnp.maximum(s, 1e-24))

        rid = (pl.program_id(0) * blk
               + lax.broadcasted_iota(jnp.int32, (blk, 1), 0))

        def sel(j, x):
            in_grp = jnp.logical_and(rid >= bnd[0, j], rid < bnd[0, j + 1])
            return jnp.where(in_grp, nrm(x[...]), 0.0)

        hv = nrm(h[...]) + sel(0, sh0) + sel(1, sh1) + sel(2, sh2)
        tv = nrm(t[...]) + sel(0, st0) + sel(1, st1) + sel(2, st2)
        rv = nrm(r[...]) + sel(0, sq0) + sel(1, sq1) + sel(2, sq2)
        d = nrm(hv) + nrm(rv) - nrm(tv)
        o[...] = jnp.sqrt(jnp.sum(d * d, axis=1, keepdims=True))

    in_specs = ([pl.BlockSpec((1, 128), lambda i: (0, 0))]
                + [pl.BlockSpec((blk, DIM), lambda i: (i, 0))] * 12)
    out_spec = pl.BlockSpec((blk, 1), lambda i: (i, 0))
    return pl.pallas_call(
        body,
        grid=(B2 // blk,),
        in_specs=in_specs,
        out_specs=out_spec,
        out_shape=jax.ShapeDtypeStruct((B2, 1), jnp.float32),
    )(bounds_tc, *g)


def kernel(ents, rels_tab, se0, se1, se2, sr0, sr1, sr2,
           heads, rels, tails, sources,
           heads_bad, rels_bad, tails_bad, sources_bad):
    ah = jnp.concatenate([heads, heads_bad]).astype(jnp.int32)
    ar = jnp.concatenate([rels, rels_bad]).astype(jnp.int32)
    at = jnp.concatenate([tails, tails_bad]).astype(jnp.int32)
    asrc = jnp.concatenate([sources, sources_bad]).astype(jnp.int32)

    key = jnp.where((asrc >= 2) & (asrc <= 4), asrc - 2, 3)
    iot = jnp.arange(B2, dtype=jnp.int32)
    onehot = (key[:, None] == jnp.arange(4, dtype=jnp.int32)[None, :])
    csum = jnp.cumsum(onehot.astype(jnp.int32), axis=0)
    n = csum[-1]
    b0 = jnp.int32(0)
    b1, b2, b3 = n[0], n[0] + n[1], n[0] + n[1] + n[2]
    off = jnp.stack([b0, b1, b2, b3])
    pos = jnp.sum(jnp.where(onehot, csum - 1 + off[None, :], 0), axis=1)
    perm = jnp.zeros((B2,), jnp.int32).at[pos].set(iot)
    bounds = jnp.stack([b0, b1, b2, b3] + [b0] * 12).astype(jnp.int32)
    bounds_tc = (jnp.zeros((1, 128), jnp.int32)
                 .at[0, 1].set(b1).at[0, 2].set(b2).at[0, 3].set(b3))

    tables = [ents, se0, se1, se2, ents, se0, se1, se2,
              rels_tab, sr0, sr1, sr2]
    gsel = [(0, None), (0, 0), (0, 1), (0, 2),
            (1, None), (1, 0), (1, 1), (1, 2),
            (2, None), (2, 0), (2, 1), (2, 2)]

    g = _sc_gather_all(tables, [ah, at, ar], gsel, bounds, perm)
    s_perm = _tc_score(g, bounds_tc)[:, 0]
    s = s_perm[pos]
    return (s[:4096], s[4096:])


# speedup vs baseline: 1.0484x; 1.0484x over previous
import dataclasses
import functools

import jax
import jax.numpy as jnp
from jax import lax
from jax.experimental import pallas as pl
from jax.experimental.pallas import tpu as pltpu
from jax.experimental.pallas import tpu_sc as plsc

NC, NS = 2, 16
NW = NC * NS
B2 = 8192
DIM = 128
CHUNK = 128
PER_W = B2 // NW
NCHUNK = PER_W // CHUNK
NBUF = 4


def _sc_gather_all(tables, idxs, gsel, bounds, perm):
    mesh = plsc.VectorSubcoreMesh(core_axis_name="c", subcore_axis_name="s")
    ng = len(tables)
    out_type = [jax.ShapeDtypeStruct((B2, DIM), jnp.float32)] * ng
    cp = pltpu.CompilerParams()
    if "needs_layout_passes" in pltpu.CompilerParams.__dataclass_fields__:
        cp = dataclasses.replace(cp, needs_layout_passes=False)

    @functools.partial(
        pl.kernel,
        mesh=mesh,
        out_type=out_type,
        compiler_params=cp,
        scratch_types=(
            [pltpu.VMEM((PER_W,), jnp.int32)] * 4
            + [pltpu.VMEM((CHUNK, DIM), jnp.float32)] * NBUF
            + [pltpu.VMEM((16,), jnp.int32)]
            + [pltpu.SemaphoreType.DMA] * (1 + 2 * NBUF)
        ),
    )
    def k(*refs):
        t_refs = refs[:ng]
        i_refs = refs[ng:ng + 3]
        b_ref = refs[ng + 3]
        p_ref = refs[ng + 4]
        o_refs = refs[ng + 5:2 * ng + 5]
        base = 2 * ng + 5
        idx_v = refs[base:base + 3]
        perm_v = refs[base + 3]
        rows_v = refs[base + 4:base + 4 + NBUF]
        bnd_s = refs[base + 4 + NBUF]
        sem_i = refs[base + 5 + NBUF]
        sem_g = refs[base + 6 + NBUF:base + 6 + 2 * NBUF]
        sem_w = refs[base + 6 + 2 * NBUF:base + 6 + 3 * NBUF]
        wid = lax.axis_index("s") * NC + lax.axis_index("c")
        base0 = wid * PER_W

        pltpu.sync_copy(b_ref, bnd_s)
        pltpu.sync_copy(p_ref.at[pl.ds(base0, PER_W)], perm_v)
        pf = [pltpu.make_async_copy(i_refs[a].at[perm_v], idx_v[a], sem_i)
              for a in range(3)]
        for d in pf:
            d.start()
        for d in pf:
            d.wait()

        bvec = bnd_s[...]
        lane = jnp.arange(16, dtype=jnp.int32)
        bs = [jnp.max(jnp.where(lane == j, bvec, 0)) for j in range(4)]

        slots = []
        for g in range(ng):
            a, j = gsel[g]
            for c in range(NCHUNK):
                slots.append((g, a, c * CHUNK, j))

        def cond_of(j, off):
            if j is None:
                return None
            p = base0 + off
            return jnp.logical_and(p < bs[j + 1], p + CHUNK > bs[j])

        def guarded(cond, fn):
            if cond is None:
                fn()
            else:
                pl.when(cond)(fn)

        conds = [cond_of(j, off) for (_, _, off, j) in slots]
        gd = [None] * NBUF
        wd = [None] * NBUF

        def issue_writeback(kk):
            s = kk % NBUF
            g, _, off, _ = slots[kk]
            d, c = gd[s]
            guarded(c, d.wait)
            w = pltpu.make_async_copy(
                rows_v[s], o_refs[g].at[pl.ds(base0 + off, CHUNK)], sem_w[s])
            guarded(c, w.start)
            wd[s] = (w, c)

        for kk in range(len(slots)):
            s = kk % NBUF
            if wd[s] is not None:
                d, c = wd[s]
                guarded(c, d.wait)
                wd[s] = None
            g, a, off, j = slots[kk]
            d = pltpu.make_async_copy(
                t_refs[g].at[idx_v[a].at[pl.ds(off, CHUNK)]],
                rows_v[s], sem_g[s])
            guarded(conds[kk], d.start)
            gd[s] = (d, conds[kk])
            if kk >= NBUF - 1:
                issue_writeback(kk - (NBUF - 1))
        for kk in range(max(0, len(slots) - (NBUF - 1)), len(slots)):
            issue_writeback(kk)
        for s in range(NBUF):
            if wd[s] is not None:
                d, c = wd[s]
                guarded(c, d.wait)

    return k(*tables, *idxs, bounds, perm)


def _tc_score(g, bounds_tc):
    blk = 1024

    def body(bnd, h, sh0, sh1, sh2, t, st0, st1, st2, r, sq0, sq1, sq2, o):
        def nrm(x):
            s = jnp.sum(x * x, axis=1, keepdims=True)
            return x * lax.rsqrt(jnp.maximum(s, 1e-24))

        rid = (pl.program_id(0) * blk
               + lax.broadcasted_iota(jnp.int32, (blk, 1), 0))
        grp = [jnp.logical_and(rid >= bnd[0, j], rid < bnd[0, j + 1])
               for j in range(3)]

        def pick(x0, x1, x2):
            z = jnp.zeros_like(x0[...])
            return nrm(jnp.where(grp[0], x0[...],
                       jnp.where(grp[1], x1[...],
                       jnp.where(grp[2], x2[...], z))))

        hv = nrm(h[...]) + pick(sh0, sh1, sh2)
        tv = nrm(t[...]) + pick(st0, st1, st2)
        rv = nrm(r[...]) + pick(sq0, sq1, sq2)
        d = nrm(hv) + nrm(rv) - nrm(tv)
        o[...] = jnp.sqrt(jnp.sum(d * d, axis=1, keepdims=True))

    in_specs = ([pl.BlockSpec((1, 128), lambda i: (0, 0))]
                + [pl.BlockSpec((blk, DIM), lambda i: (i, 0))] * 12)
    out_spec = pl.BlockSpec((blk, 1), lambda i: (i, 0))
    return pl.pallas_call(
        body,
        grid=(B2 // blk,),
        in_specs=in_specs,
        out_specs=out_spec,
        out_shape=jax.ShapeDtypeStruct((B2, 1), jnp.float32),
    )(bounds_tc, *g)


def kernel(ents, rels_tab, se0, se1, se2, sr0, sr1, sr2,
           heads, rels, tails, sources,
           heads_bad, rels_bad, tails_bad, sources_bad):
    ah = jnp.concatenate([heads, heads_bad]).astype(jnp.int32)
    ar = jnp.concatenate([rels, rels_bad]).astype(jnp.int32)
    at = jnp.concatenate([tails, tails_bad]).astype(jnp.int32)
    asrc = jnp.concatenate([sources, sources_bad]).astype(jnp.int32)

    key = jnp.where((asrc >= 2) & (asrc <= 4), asrc - 2, 3)
    iot = jnp.arange(B2, dtype=jnp.int32)
    onehot = (key[:, None] == jnp.arange(4, dtype=jnp.int32)[None, :])
    csum = jnp.cumsum(onehot.astype(jnp.int32), axis=0)
    n = csum[-1]
    b0 = jnp.int32(0)
    b1, b2, b3 = n[0], n[0] + n[1], n[0] + n[1] + n[2]
    off = jnp.stack([b0, b1, b2, b3])
    pos = jnp.sum(jnp.where(onehot, csum - 1 + off[None, :], 0), axis=1)
    perm = jnp.zeros((B2,), jnp.int32).at[pos].set(iot)
    bounds = jnp.stack([b0, b1, b2, b3] + [b0] * 12).astype(jnp.int32)
    bounds_tc = (jnp.zeros((1, 128), jnp.int32)
                 .at[0, 1].set(b1).at[0, 2].set(b2).at[0, 3].set(b3))

    tables = [ents, se0, se1, se2, ents, se0, se1, se2,
              rels_tab, sr0, sr1, sr2]
    gsel = [(0, None), (0, 0), (0, 1), (0, 2),
            (1, None), (1, 0), (1, 1), (1, 2),
            (2, None), (2, 0), (2, 1), (2, 2)]

    g = _sc_gather_all(tables, [ah, at, ar], gsel, bounds, perm)
    s_perm = _tc_score(g, bounds_tc)[:, 0]
    s = s_perm[pos]
    return (s[:4096], s[4096:])


# speedup vs baseline: 1.0505x; 1.0019x over previous
import dataclasses
import functools

import jax
import jax.numpy as jnp
from jax import lax
from jax.experimental import pallas as pl
from jax.experimental.pallas import tpu as pltpu
from jax.experimental.pallas import tpu_sc as plsc

NC, NS = 2, 16
NW = NC * NS
B2 = 8192
DIM = 128
CHUNK = 128
PER_W = B2 // NW
NCHUNK = PER_W // CHUNK
NBUF = 4


def _sc_gather_all(tables, idxs, gsel, bounds, perm, half, nhalf):
    mesh = plsc.VectorSubcoreMesh(core_axis_name="c", subcore_axis_name="s")
    ng = len(tables)
    hb = B2 // nhalf
    per_w = hb // NW
    nchunk = max(1, per_w // CHUNK)
    cw = per_w // nchunk
    out_type = [jax.ShapeDtypeStruct((hb, DIM), jnp.float32)] * ng
    cp = pltpu.CompilerParams()
    if "needs_layout_passes" in pltpu.CompilerParams.__dataclass_fields__:
        cp = dataclasses.replace(cp, needs_layout_passes=False)

    @functools.partial(
        pl.kernel,
        mesh=mesh,
        out_type=out_type,
        compiler_params=cp,
        scratch_types=(
            [pltpu.VMEM((per_w,), jnp.int32)] * 4
            + [pltpu.VMEM((CHUNK, DIM), jnp.float32)] * NBUF
            + [pltpu.VMEM((16,), jnp.int32)]
            + [pltpu.SemaphoreType.DMA] * (1 + 2 * NBUF)
        ),
    )
    def k(*refs):
        t_refs = refs[:ng]
        i_refs = refs[ng:ng + 3]
        b_ref = refs[ng + 3]
        p_ref = refs[ng + 4]
        o_refs = refs[ng + 5:2 * ng + 5]
        base = 2 * ng + 5
        idx_v = refs[base:base + 3]
        perm_v = refs[base + 3]
        rows_v = refs[base + 4:base + 4 + NBUF]
        bnd_s = refs[base + 4 + NBUF]
        sem_i = refs[base + 5 + NBUF]
        sem_g = refs[base + 6 + NBUF:base + 6 + 2 * NBUF]
        sem_w = refs[base + 6 + 2 * NBUF:base + 6 + 3 * NBUF]
        wid = lax.axis_index("s") * NC + lax.axis_index("c")
        base0 = wid * per_w
        gbase = half * hb + base0

        pltpu.sync_copy(b_ref, bnd_s)
        pltpu.sync_copy(p_ref.at[pl.ds(gbase, per_w)], perm_v)
        pf = [pltpu.make_async_copy(i_refs[a].at[perm_v], idx_v[a], sem_i)
              for a in range(3)]
        for d in pf:
            d.start()
        for d in pf:
            d.wait()

        bvec = bnd_s[...]
        lane = jnp.arange(16, dtype=jnp.int32)
        bs = [jnp.max(jnp.where(lane == j, bvec, 0)) for j in range(4)]

        slots = []
        for g in range(ng):
            a, j = gsel[g]
            for c in range(nchunk):
                slots.append((g, a, c * CHUNK, j))

        def cond_of(j, off):
            if j is None:
                return None
            p = gbase + off
            return jnp.logical_and(p < bs[j + 1], p + CHUNK > bs[j])

        def guarded(cond, fn):
            if cond is None:
                fn()
            else:
                pl.when(cond)(fn)

        conds = [cond_of(j, off) for (_, _, off, j) in slots]
        gd = [None] * NBUF
        wd = [None] * NBUF

        def issue_writeback(kk):
            s = kk % NBUF
            g, _, off, _ = slots[kk]
            d, c = gd[s]
            guarded(c, d.wait)
            w = pltpu.make_async_copy(
                rows_v[s], o_refs[g].at[pl.ds(base0 + off, CHUNK)], sem_w[s])
            guarded(c, w.start)
            wd[s] = (w, c)

        for kk in range(len(slots)):
            s = kk % NBUF
            if wd[s] is not None:
                d, c = wd[s]
                guarded(c, d.wait)
                wd[s] = None
            g, a, off, j = slots[kk]
            d = pltpu.make_async_copy(
                t_refs[g].at[idx_v[a].at[pl.ds(off, CHUNK)]],
                rows_v[s], sem_g[s])
            guarded(conds[kk], d.start)
            gd[s] = (d, conds[kk])
            if kk >= NBUF - 1:
                issue_writeback(kk - (NBUF - 1))
        for kk in range(max(0, len(slots) - (NBUF - 1)), len(slots)):
            issue_writeback(kk)
        for s in range(NBUF):
            if wd[s] is not None:
                d, c = wd[s]
                guarded(c, d.wait)

    return k(*tables, *idxs, bounds, perm)


def _tc_score(g, bounds_tc, half, nhalf):
    hb = B2 // nhalf
    blk = 1024

    def body(bnd, h, sh0, sh1, sh2, t, st0, st1, st2, r, sq0, sq1, sq2, o):
        def nrm(x):
            s = jnp.sum(x * x, axis=1, keepdims=True)
            return x * lax.rsqrt(jnp.maximum(s, 1e-24))

        rid = (half * hb + pl.program_id(0) * blk
               + lax.broadcasted_iota(jnp.int32, (blk, 1), 0))
        grp = [jnp.logical_and(rid >= bnd[0, j], rid < bnd[0, j + 1])
               for j in range(3)]

        def pick(x0, x1, x2):
            z = jnp.zeros_like(x0[...])
            return nrm(jnp.where(grp[0], x0[...],
                       jnp.where(grp[1], x1[...],
                       jnp.where(grp[2], x2[...], z))))

        hv = nrm(h[...]) + pick(sh0, sh1, sh2)
        tv = nrm(t[...]) + pick(st0, st1, st2)
        rv = nrm(r[...]) + pick(sq0, sq1, sq2)
        d = nrm(hv) + nrm(rv) - nrm(tv)
        o[...] = jnp.sqrt(jnp.sum(d * d, axis=1, keepdims=True))

    in_specs = ([pl.BlockSpec((1, 128), lambda i: (0, 0))]
                + [pl.BlockSpec((blk, DIM), lambda i: (i, 0))] * 12)
    out_spec = pl.BlockSpec((blk, 1), lambda i: (i, 0))
    return pl.pallas_call(
        body,
        grid=(hb // blk,),
        in_specs=in_specs,
        out_specs=out_spec,
        out_shape=jax.ShapeDtypeStruct((hb, 1), jnp.float32),
    )(bounds_tc, *g)


def kernel(ents, rels_tab, se0, se1, se2, sr0, sr1, sr2,
           heads, rels, tails, sources,
           heads_bad, rels_bad, tails_bad, sources_bad):
    ah = jnp.concatenate([heads, heads_bad]).astype(jnp.int32)
    ar = jnp.concatenate([rels, rels_bad]).astype(jnp.int32)
    at = jnp.concatenate([tails, tails_bad]).astype(jnp.int32)
    asrc = jnp.concatenate([sources, sources_bad]).astype(jnp.int32)

    key = jnp.where((asrc >= 2) & (asrc <= 4), asrc - 2, 3)
    iot = jnp.arange(B2, dtype=jnp.int32)
    onehot = (key[:, None] == jnp.arange(4, dtype=jnp.int32)[None, :])
    csum = jnp.cumsum(onehot.astype(jnp.int32), axis=0)
    n = csum[-1]
    b0 = jnp.int32(0)
    b1, b2, b3 = n[0], n[0] + n[1], n[0] + n[1] + n[2]
    off = jnp.stack([b0, b1, b2, b3])
    pos = jnp.sum(jnp.where(onehot, csum - 1 + off[None, :], 0), axis=1)
    perm = jnp.zeros((B2,), jnp.int32).at[pos].set(iot)
    bounds = jnp.stack([b0, b1, b2, b3] + [b0] * 12).astype(jnp.int32)
    bounds_tc = (jnp.zeros((1, 128), jnp.int32)
                 .at[0, 1].set(b1).at[0, 2].set(b2).at[0, 3].set(b3))

    tables = [ents, se0, se1, se2, ents, se0, se1, se2,
              rels_tab, sr0, sr1, sr2]
    gsel = [(0, None), (0, 0), (0, 1), (0, 2),
            (1, None), (1, 0), (1, 1), (1, 2),
            (2, None), (2, 0), (2, 1), (2, 2)]

    NH = 2
    halves = []
    for hf in range(NH):
        gh = _sc_gather_all(tables, [ah, at, ar], gsel, bounds, perm, hf, NH)
        halves.append(_tc_score(gh, bounds_tc, hf, NH)[:, 0])
    s_perm = jnp.concatenate(halves)
    s = s_perm[pos]
    return (s[:4096], s[4096:])
